# baseline (speedup 1.0000x reference)
import dataclasses
import functools

import jax
import jax.numpy as jnp
from jax import lax
from jax.experimental import pallas as pl
from jax.experimental.pallas import tpu as pltpu
from jax.experimental.pallas import tpu_sc as plsc

N_NODES_ = 10000
D_ = 128
N_EDGES_ = 320000

NC = 2
NS = 16
NW = NC * NS
LANES = 16

PER_W = N_EDGES_ // NW
E = 80
NCHUNK = PER_W // E


def _gae_decode(z, src_idx, dst_idx):
    mesh = plsc.VectorSubcoreMesh(core_axis_name="c", subcore_axis_name="s")

    cp = pltpu.CompilerParams()
    if "needs_layout_passes" in pltpu.CompilerParams.__dataclass_fields__:
        cp = dataclasses.replace(cp, needs_layout_passes=False)

    @functools.partial(
        pl.kernel,
        compiler_params=cp,
        out_type=jax.ShapeDtypeStruct((N_EDGES_,), jnp.float32),
        mesh=mesh,
        scratch_types=[
            pltpu.VMEM_SHARED((N_NODES_, D_), jnp.float32),
            pltpu.VMEM((E,), jnp.int32),
            pltpu.VMEM((E,), jnp.int32),
            pltpu.VMEM((E,), jnp.int32),
            pltpu.VMEM((E,), jnp.int32),
            pltpu.VMEM((E, D_), jnp.float32),
            pltpu.VMEM((E, D_), jnp.float32),
            pltpu.VMEM((E, D_), jnp.float32),
            pltpu.VMEM((E, D_), jnp.float32),
            pltpu.VMEM((E,), jnp.float32),
            pltpu.VMEM((E,), jnp.float32),
            pltpu.VMEM((LANES, LANES + 1), jnp.float32),
            pltpu.SemaphoreType.DMA,
            pltpu.SemaphoreType.DMA,
            pltpu.SemaphoreType.DMA,
            pltpu.SemaphoreType.DMA,
            pltpu.SemaphoreType.DMA,
            pltpu.SemaphoreType.DMA,
        ],
    )
    def kern(z_hbm, si_hbm, di_hbm, out_hbm, z_spm,
             si0, di0, si1, di1, srows0, drows0, srows1, drows1,
             out0, out1, part,
             gsem0, gsem1, isem0, isem1, osem0, osem1):
        sid = lax.axis_index("s")
        wid = sid * NC + lax.axis_index("c")
        wbase = wid * PER_W

        @pl.when(sid < NS - 1)
        def _():
            pltpu.sync_copy(z_hbm.at[pl.ds(sid * 624, 624)],
                            z_spm.at[pl.ds(sid * 624, 624)])

        @pl.when(sid == NS - 1)
        def _():
            pltpu.sync_copy(z_hbm.at[pl.ds(15 * 624, 640)],
                            z_spm.at[pl.ds(15 * 624, 640)])

        plsc.subcore_barrier()

        def fire_idx(j, si_v, di_v, isem):
            pltpu.async_copy(si_hbm.at[pl.ds(wbase + j * E, E)], si_v, isem)
            pltpu.async_copy(di_hbm.at[pl.ds(wbase + j * E, E)], di_v, isem)

        def wait_idx(j, si_v, di_v, isem):
            pltpu.make_async_copy(
                si_hbm.at[pl.ds(wbase + j * E, E)], si_v, isem).wait()
            pltpu.make_async_copy(
                di_hbm.at[pl.ds(wbase + j * E, E)], di_v, isem).wait()

        def fire_rows(si_v, di_v, sb, db, gsem):
            pltpu.async_copy(z_spm.at[si_v], sb, gsem)
            pltpu.async_copy(z_spm.at[di_v], db, gsem)

        def wait_rows(si_v, di_v, sb, db, gsem):
            pltpu.make_async_copy(z_spm.at[si_v], sb, gsem).wait()
            pltpu.make_async_copy(z_spm.at[di_v], db, gsem).wait()

        def fire_out(j, out_v, osem):
            pltpu.async_copy(out_v, out_hbm.at[pl.ds(wbase + j * E, E)], osem)

        def wait_out(j, out_v, osem):
            pltpu.make_async_copy(
                out_v, out_hbm.at[pl.ds(wbase + j * E, E)], osem).wait()

        def compute(sb, db, out_v):
            @pl.loop(0, E, step=LANES)
            def _grp(g):
                for el in range(LANES):
                    acc = (sb[g + el, pl.ds(0, LANES)]
                           * db[g + el, pl.ds(0, LANES)])
                    for c in range(1, D_ // LANES):
                        acc = acc + (sb[g + el, pl.ds(c * LANES, LANES)]
                                     * db[g + el, pl.ds(c * LANES, LANES)])
                    part[el, pl.ds(0, LANES)] = acc
                rows = lax.iota(jnp.int32, LANES)
                tot = plsc.load_gather(
                    part, [rows, jnp.zeros((LANES,), jnp.int32)])
                for col in range(1, LANES):
                    tot = tot + plsc.load_gather(
                        part, [rows, jnp.full((LANES,), col, jnp.int32)])
                out_v[pl.ds(g, LANES)] = 1.0 / (1.0 + jnp.exp(-tot))

        fire_idx(0, si0, di0, isem0)
        fire_idx(1, si1, di1, isem1)
        wait_idx(0, si0, di0, isem0)
        fire_rows(si0, di0, srows0, drows0, gsem0)
        wait_idx(1, si1, di1, isem1)
        fire_rows(si1, di1, srows1, drows1, gsem1)

        @pl.loop(0, NCHUNK - 1, step=2)
        def _pair(j):
            wait_rows(si0, di0, srows0, drows0, gsem0)
            fire_idx(j + 2, si0, di0, isem0)

            @pl.when(j >= 2)
            def _():
                wait_out(j, out0, osem0)

            compute(srows0, drows0, out0)
            fire_out(j, out0, osem0)
            wait_idx(j + 2, si0, di0, isem0)
            fire_rows(si0, di0, srows0, drows0, gsem0)

            wait_rows(si1, di1, srows1, drows1, gsem1)

            @pl.when(j + 3 < NCHUNK)
            def _():
                fire_idx(j + 3, si1, di1, isem1)

            @pl.when(j >= 2)
            def _():
                wait_out(j + 1, out1, osem1)

            compute(srows1, drows1, out1)
            fire_out(j + 1, out1, osem1)

            @pl.when(j + 3 < NCHUNK)
            def _():
                wait_idx(j + 3, si1, di1, isem1)
                fire_rows(si1, di1, srows1, drows1, gsem1)

        wait_rows(si0, di0, srows0, drows0, gsem0)
        wait_out(NCHUNK - 3, out0, osem0)
        compute(srows0, drows0, out0)
        fire_out(NCHUNK - 1, out0, osem0)

        wait_out(NCHUNK - 1, out0, osem0)
        wait_out(NCHUNK - 2, out1, osem1)

    return kern(z, src_idx, dst_idx)


@jax.jit
def kernel(z, edge_index):
    src = edge_index[0].astype(jnp.int32)
    dst = edge_index[1].astype(jnp.int32)
    return _gae_decode(z, src, dst)
